# baseline (speedup 1.0000x reference)
import functools
import math

import jax
import jax.numpy as jnp
from jax import lax
from jax.experimental import pallas as pl
from jax.experimental.pallas import tpu as pltpu
from jax.experimental.pallas import tpu_sc as plsc

_K = 1000
_B = 16384
_EPS = 0.1
_CONF = 1.0 - _EPS
_FILL = _EPS / (_K - 1)
_CONST = _B * ((_K - 1) * _FILL * math.log(_FILL) + _CONF * math.log(_CONF))
_SCALE = 1.0 / (_B * _K)

_R0 = 800
_RSC = _K - _R0

_BLK = 2048
_NBLK = _B // _BLK

_NC = 2
_NS = 16
_NW = _NC * _NS
_SW = _B // _NW
_G = _SW // 128


@functools.partial(
    pl.kernel,
    mesh=plsc.VectorSubcoreMesh(core_axis_name="c", subcore_axis_name="s"),
    out_type=jax.ShapeDtypeStruct((_NW, 2, 16), jnp.float32),
    scratch_types=[
        pltpu.VMEM((_SW,), jnp.int32),
        pltpu.VMEM((_RSC, 128), jnp.float32),
        pltpu.VMEM((_RSC, 128), jnp.float32),
        pltpu.VMEM((2, 16), jnp.float32),
        pltpu.SemaphoreType.DMA,
        pltpu.SemaphoreType.DMA,
    ],
)
def _sc_partials(predt_hbm, tgt_hbm, out_hbm, tgt_v, buf0, buf1, acc_v, sem0, sem1):
    wid = lax.axis_index("s") * _NC + lax.axis_index("c")
    cbase = pl.multiple_of(wid * _SW, 128)
    pltpu.sync_copy(tgt_hbm.at[pl.ds(cbase, _SW)], tgt_v)

    bufs = (buf0, buf1)
    sems = (sem0, sem1)
    zero16 = jnp.zeros((16,), jnp.float32)
    iota16 = lax.iota(jnp.int32, 16)

    acc_v[0] = zero16
    acc_v[1] = zero16

    def _start(g, b):
        c0 = pl.multiple_of(cbase + g * 128, 128)
        pltpu.async_copy(
            predt_hbm.at[pl.ds(_R0, _RSC), pl.ds(c0, 128)], bufs[b], sems[b]
        )

    def _wait(b):
        pltpu.make_async_copy(
            predt_hbm.at[pl.ds(_R0, _RSC), pl.ds(0, 128)], bufs[b], sems[b]
        ).wait()

    def _process(g, b):
        buf = bufs[b]

        def _row(r, accs):
            return tuple(accs[j] + buf[r, pl.ds(j * 16, 16)] for j in range(8))

        accs = lax.fori_loop(0, _RSC, _row, tuple([zero16] * 8))
        acc_s = accs[0]
        for j in range(1, 8):
            acc_s = acc_s + accs[j]
        acc_v[0] = acc_v[0] + acc_s

        gaccs = [zero16] * 8
        for cc in range(8):
            t16 = tgt_v[pl.ds(g * 128 + cc * 16, 16)]
            for i in range(16):
                ti = t16[i]
                valid_f = jnp.where(ti >= _R0, jnp.float32(1.0), jnp.float32(0.0))
                tloc = jnp.maximum(ti - _R0, 0)
                chunkv = buf[tloc, pl.ds(cc * 16, 16)]
                pick = jnp.where(iota16 == i, chunkv, jnp.float32(0.0)) * valid_f
                gaccs[i % 8] = gaccs[i % 8] + pick
        acc_g = gaccs[0]
        for j in range(1, 8):
            acc_g = acc_g + gaccs[j]
        acc_v[1] = acc_v[1] + acc_g

    _start(0, 0)
    _start(1, 1)
    for g in range(_G):
        b = g % 2
        _wait(b)
        _process(g, b)
        if g + 2 < _G:
            _start(g + 2, b)

    pltpu.sync_copy(acc_v, out_hbm.at[wid])


def _tc_body(tgt_ref, predt_ref, out_ref):
    i = pl.program_id(0)
    x = predt_ref[...]
    tgt = tgt_ref[0]
    psum = jnp.sum(x)
    rows = lax.broadcasted_iota(jnp.int32, (_R0, _BLK), 0)
    mask = rows == tgt
    gsum = jnp.sum(jnp.where(mask, x, 0.0))

    @pl.when(i == 0)
    def _init():
        out_ref[0, 0] = jnp.float32(0.0)
        out_ref[0, 1] = jnp.float32(0.0)

    out_ref[0, 0] += psum
    out_ref[0, 1] += gsum


def _combine_body(p_ref, t_ref, out_ref):
    s = jnp.sum(p_ref[:, 0, :]) + t_ref[0, 0]
    g = jnp.sum(p_ref[:, 1, :]) + t_ref[0, 1]
    out_ref[0, 0] = (
        jnp.float32(_CONST) - jnp.float32(_FILL) * s - jnp.float32(_CONF - _FILL) * g
    ) * jnp.float32(_SCALE)


def kernel(pred, target):
    predt = pred.T
    tgt = target.astype(jnp.int32)
    sc_part = _sc_partials(predt, tgt)
    tgt3 = tgt.reshape(_NBLK, 1, _BLK)
    tc_part = pl.pallas_call(
        _tc_body,
        grid=(_NBLK,),
        in_specs=[
            pl.BlockSpec((1, 1, _BLK), lambda i: (i, 0, 0)),
            pl.BlockSpec((_R0, _BLK), lambda i: (0, i)),
        ],
        out_specs=pl.BlockSpec((1, 2), lambda i: (0, 0), memory_space=pltpu.SMEM),
        out_shape=jax.ShapeDtypeStruct((1, 2), jnp.float32),
    )(tgt3, predt)
    out = pl.pallas_call(
        _combine_body,
        in_specs=[
            pl.BlockSpec(memory_space=pltpu.VMEM),
            pl.BlockSpec(memory_space=pltpu.SMEM),
        ],
        out_specs=pl.BlockSpec(memory_space=pltpu.SMEM),
        out_shape=jax.ShapeDtypeStruct((1, 1), jnp.float32),
    )(sc_part, tc_part)
    return out.reshape(())


# speedup vs baseline: 1.5519x; 1.5519x over previous
import math

import jax
import jax.numpy as jnp
from jax import lax
from jax.experimental import pallas as pl
from jax.experimental.pallas import tpu as pltpu

_K = 1000
_B = 16384
_EPS = 0.1
_CONF = 1.0 - _EPS
_FILL = _EPS / (_K - 1)
_CONST = _B * ((_K - 1) * _FILL * math.log(_FILL) + _CONF * math.log(_CONF))
_SCALE = 1.0 / (_B * _K)

_BLK = 2048
_NBLK = _B // _BLK


def _loss_body(tgt_ref, predt_ref, out_ref):
    i = pl.program_id(0)
    x = predt_ref[...]
    tgt = tgt_ref[0]
    psum = jnp.sum(x)
    rows = lax.broadcasted_iota(jnp.int32, (_K, _BLK), 0)
    mask = rows == tgt
    gsum = jnp.sum(jnp.where(mask, x, 0.0))
    contrib = (-_FILL * psum - (_CONF - _FILL) * gsum) * _SCALE

    @pl.when(i == 0)
    def _init():
        out_ref[0, 0] = jnp.float32(_CONST * _SCALE)

    out_ref[0, 0] += contrib


def kernel(pred, target):
    predt = pred.T
    tgt3 = target.astype(jnp.int32).reshape(_NBLK, 1, _BLK)
    out = pl.pallas_call(
        _loss_body,
        grid=(_NBLK,),
        in_specs=[
            pl.BlockSpec((1, 1, _BLK), lambda i: (i, 0, 0)),
            pl.BlockSpec((_K, _BLK), lambda i: (0, i)),
        ],
        out_specs=pl.BlockSpec((1, 1), lambda i: (0, 0), memory_space=pltpu.SMEM),
        out_shape=jax.ShapeDtypeStruct((1, 1), jnp.float32),
    )(tgt3, predt)
    return out.reshape(())
